# Initial kernel scaffold; baseline (speedup 1.0000x reference)
import jax
import jax.numpy as jnp
from jax.experimental import pallas as pl


def kernel(x, edge_index, batch, W1l, b1l, W1r, W2l, b2l, W2r, Wf1, bf1, Wf2, bf2):
    raise NotImplementedError("write your pallas kernel here")


# speedup vs baseline: 4.6554x; 4.6554x over previous
import functools

import jax
import jax.numpy as jnp
from jax import lax
from jax.experimental import pallas as pl
from jax.experimental.pallas import tpu as pltpu
from jax.experimental.pallas import tpu_sc as plsc

N = 10000
E = 320000
D = 128
H = 128
G = 16
C = 2

NPAD = 10240
NW = 32
CH = 80
NCH = 125
EPW = CH * NCH
RPT = NPAD // 16
BLK = 512
NB = NPAD // BLK


def _make_sc_agg(W):
    out_type = jax.ShapeDtypeStruct((2 * NPAD, W), jnp.float32)
    scratch = [
        pltpu.VMEM((CH,), jnp.int32),
        pltpu.VMEM((CH,), jnp.int32),
        pltpu.VMEM((CH, W), jnp.float32),
        pltpu.VMEM_SHARED((NPAD, W), jnp.float32),
        pltpu.SemaphoreType.DMA,
    ]

    mesh = plsc.VectorSubcoreMesh(core_axis_name="c", subcore_axis_name="s")

    def body(x_hbm, src_hbm, dst_hbm, z2_hbm, acc_out, sidx, didx, rows,
             acc_sh, sem):
        c = lax.axis_index("c")
        s = lax.axis_index("s")
        wid = s * 2 + c

        pltpu.sync_copy(z2_hbm.at[pl.ds(0, CH)], rows)
        for k in range(RPT // CH):
            pltpu.sync_copy(rows, acc_sh.at[pl.ds(s * RPT + k * CH, CH)])
        plsc.subcore_barrier()

        base0 = wid * EPW

        def step(i, carry):
            b = base0 + i * CH
            pltpu.sync_copy(src_hbm.at[pl.ds(b, CH)], sidx)
            pltpu.sync_copy(dst_hbm.at[pl.ds(b, CH)], didx)
            pltpu.async_copy(x_hbm.at[sidx], rows, sem).wait()
            pltpu.sync_copy(rows, acc_sh.at[didx], add=True)
            return carry

        lax.fori_loop(0, NCH, step, 0)
        plsc.subcore_barrier()

        for k in range(RPT // CH):
            pltpu.sync_copy(acc_sh.at[pl.ds(s * RPT + k * CH, CH)], rows)
            pltpu.sync_copy(rows, acc_out.at[pl.ds(c * NPAD + s * RPT + k * CH, CH)])

    return pl.kernel(body, out_type=out_type, mesh=mesh,
                     scratch_types=scratch)


def _tc_layer1_body(acc0, acc1, cnt0, cnt1, x, wl, wr, b, out):
    cnt = cnt0[:, 0:1] + cnt1[:, 0:1]
    agg = (acc0[...] + acc1[...]) / jnp.maximum(cnt, 1.0)
    h = (jnp.dot(agg, wl[...], preferred_element_type=jnp.float32)
         + jnp.dot(x[...], wr[...], preferred_element_type=jnp.float32)
         + b[...])
    out[...] = jnp.maximum(h, 0.0)


def _tc_layer2_body(acc0, acc1, cnt0, cnt1, h1, batch, wl, wr, b,
                    wf1, bf1, wf2, bf2, out, pool_acc, n_acc):
    i = pl.program_id(0)

    @pl.when(i == 0)
    def _():
        pool_acc[...] = jnp.zeros_like(pool_acc)
        n_acc[...] = jnp.zeros_like(n_acc)

    cnt = cnt0[:, 0:1] + cnt1[:, 0:1]
    agg = (acc0[...] + acc1[...]) / jnp.maximum(cnt, 1.0)
    h = (jnp.dot(agg, wl[...], preferred_element_type=jnp.float32)
         + jnp.dot(h1[...], wr[...], preferred_element_type=jnp.float32)
         + b[...])
    h = jnp.maximum(h, 0.0)

    bt = batch[0, 0, :]
    gid = lax.broadcasted_iota(jnp.int32, (G, BLK), 0)
    onehot = (gid == bt[None, :]).astype(jnp.float32)
    pool_acc[...] += jnp.dot(onehot, h, preferred_element_type=jnp.float32)
    n_acc[...] += jnp.sum(onehot, axis=1, keepdims=True)

    @pl.when(i == NB - 1)
    def _():
        pooled = pool_acc[...] / jnp.maximum(n_acc[...], 1.0)
        hid = jnp.maximum(
            jnp.dot(pooled, wf1[...], preferred_element_type=jnp.float32)
            + bf1[...], 0.0)
        out[...] = (jnp.dot(hid, wf2[...], preferred_element_type=jnp.float32)
                    + bf2[...])


_row_spec = pl.BlockSpec((BLK, D), lambda i: (i, 0))
_cnt_spec = pl.BlockSpec((BLK, 16), lambda i: (i, 0))
_full = lambda shape: pl.BlockSpec(shape, lambda i: tuple(0 for _ in shape))

_tc_layer1 = pl.pallas_call(
    _tc_layer1_body,
    grid=(NB,),
    in_specs=[_row_spec, _row_spec, _cnt_spec, _cnt_spec, _row_spec,
              _full((D, H)), _full((D, H)), _full((1, H))],
    out_specs=_row_spec,
    out_shape=jax.ShapeDtypeStruct((NPAD, H), jnp.float32),
    compiler_params=pltpu.CompilerParams(
        dimension_semantics=("arbitrary",)),
)

_tc_layer2 = pl.pallas_call(
    _tc_layer2_body,
    grid=(NB,),
    in_specs=[_row_spec, _row_spec, _cnt_spec, _cnt_spec, _row_spec,
              pl.BlockSpec((1, 1, BLK), lambda i: (i, 0, 0)),
              _full((H, H)), _full((H, H)), _full((1, H)),
              _full((H, H // 2)), _full((1, H // 2)),
              _full((H // 2, 128)), _full((1, 128))],
    out_specs=_full((G, 128)),
    out_shape=jax.ShapeDtypeStruct((G, 128), jnp.float32),
    scratch_shapes=[pltpu.VMEM((G, H), jnp.float32),
                    pltpu.VMEM((G, H), jnp.float32)],
    compiler_params=pltpu.CompilerParams(
        dimension_semantics=("arbitrary",)),
)

_sc_agg = _make_sc_agg(D)


def _sc_cnt_body(dst_hbm, z_hbm, ones_hbm, cnt_out, didx, ones_v, acc_sh):
    c = lax.axis_index("c")
    s = lax.axis_index("s")
    wid = s * 2 + c

    pltpu.sync_copy(z_hbm.at[pl.ds(0, CH)], ones_v)
    for k in range(RPT // CH):
        pltpu.sync_copy(ones_v, acc_sh.at[pl.ds(s * RPT + k * CH, CH)])
    pltpu.sync_copy(ones_hbm, ones_v)
    plsc.subcore_barrier()

    base0 = wid * EPW

    def step(i, carry):
        pltpu.sync_copy(dst_hbm.at[pl.ds(base0 + i * CH, CH)], didx)
        pltpu.sync_copy(ones_v, acc_sh.at[didx], add=True)
        return carry

    lax.fori_loop(0, NCH, step, 0)
    plsc.subcore_barrier()

    for k in range(RPT // CH):
        pltpu.sync_copy(acc_sh.at[pl.ds(s * RPT + k * CH, CH)], ones_v)
        pltpu.sync_copy(ones_v, cnt_out.at[pl.ds(c * NPAD + s * RPT + k * CH, CH)])


_sc_cnt = pl.kernel(
    _sc_cnt_body,
    out_type=jax.ShapeDtypeStruct((2 * NPAD, D), jnp.float32),
    mesh=plsc.VectorSubcoreMesh(core_axis_name="c", subcore_axis_name="s"),
    scratch_types=[
        pltpu.VMEM((CH,), jnp.int32),
        pltpu.VMEM((CH, D), jnp.float32),
        pltpu.VMEM_SHARED((NPAD, D), jnp.float32),
    ],
)


def kernel(x, edge_index, batch, W1l, b1l, W1r, W2l, b2l, W2r,
           Wf1, bf1, Wf2, bf2):
    f32 = jnp.float32
    src_p = edge_index[0]
    dst_p = edge_index[1]
    x_p = jnp.pad(x, ((0, NPAD - N), (0, 0)))
    batch_p = jnp.pad(batch, (0, NPAD - N), constant_values=G)
    batch3 = batch_p.reshape(NB, 1, BLK)

    z2 = jnp.zeros((NPAD, D), f32)
    ones_rows = jnp.ones((CH, D), f32)

    acc = _sc_agg(x_p, src_p, dst_p, z2)
    cntw = _sc_cnt(dst_p, z2, ones_rows)
    cnt0, cnt1 = cntw[:NPAD, :16], cntw[NPAD:, :16]
    h1 = _tc_layer1(acc[:NPAD], acc[NPAD:], cnt0, cnt1, x_p,
                    W1l.T, W1r.T, b1l.reshape(1, H))

    acc2 = _sc_agg(h1, src_p, dst_p, z2)
    wf2_pad = jnp.zeros((H // 2, 128), f32).at[:, :C].set(Wf2.T)
    bf2_pad = jnp.zeros((1, 128), f32).at[0, :C].set(bf2)
    out_pad = _tc_layer2(acc2[:NPAD], acc2[NPAD:], cnt0, cnt1, h1, batch3,
                         W2l.T, W2r.T, b2l.reshape(1, H),
                         Wf1.T, bf1.reshape(1, H // 2), wf2_pad, bf2_pad)
    return out_pad[:, :C]


# speedup vs baseline: 6.6549x; 1.4295x over previous
import functools

import jax
import jax.numpy as jnp
from jax import lax
from jax.experimental import pallas as pl
from jax.experimental.pallas import tpu as pltpu
from jax.experimental.pallas import tpu_sc as plsc

N = 10000
E = 320000
D = 128
H = 128
G = 16
C = 2

NPAD = 10240
NW = 32
CH = 80
NCH = 125
EPW = CH * NCH
RPT = NPAD // 16
BLK = 512
NB = NPAD // BLK


def _make_sc_agg(W):
    out_type = jax.ShapeDtypeStruct((2 * NPAD, W), jnp.float32)
    scratch = [
        pltpu.VMEM((CH,), jnp.int32),
        pltpu.VMEM((CH,), jnp.int32),
        pltpu.VMEM((CH,), jnp.int32),
        pltpu.VMEM((CH,), jnp.int32),
        pltpu.VMEM((CH, W), jnp.float32),
        pltpu.VMEM((CH, W), jnp.float32),
        pltpu.VMEM_SHARED((NPAD, W), jnp.float32),
        pltpu.SemaphoreType.DMA,
        pltpu.SemaphoreType.DMA,
    ]

    mesh = plsc.VectorSubcoreMesh(core_axis_name="c", subcore_axis_name="s")

    def body(x_hbm, src_hbm, dst_hbm, z2_hbm, acc_out, sidx0, sidx1,
             didx0, didx1, rows0, rows1, acc_sh, sem0, sem1):
        c = lax.axis_index("c")
        s = lax.axis_index("s")
        wid = s * 2 + c

        pltpu.sync_copy(z2_hbm.at[pl.ds(0, CH)], rows0)
        for k in range(RPT // CH):
            pltpu.sync_copy(rows0, acc_sh.at[pl.ds(s * RPT + k * CH, CH)])
        plsc.subcore_barrier()

        base0 = wid * EPW
        bufs = ((sidx0, didx0, rows0, sem0), (sidx1, didx1, rows1, sem1))

        pltpu.sync_copy(src_hbm.at[pl.ds(base0, CH)], sidx0)
        pltpu.sync_copy(dst_hbm.at[pl.ds(base0, CH)], didx0)
        pltpu.make_async_copy(x_hbm.at[sidx0], rows0, sem0).start()

        def halfstep(i, cur, nxt):
            csi, cdi, crow, csem = cur
            nsi, ndi, nrow, nsem = nxt
            b = base0 + (i + 1) * CH
            pltpu.sync_copy(src_hbm.at[pl.ds(b, CH)], nsi)
            pltpu.sync_copy(dst_hbm.at[pl.ds(b, CH)], ndi)
            pltpu.make_async_copy(x_hbm.at[nsi], nrow, nsem).start()
            pltpu.make_async_copy(x_hbm.at[csi], crow, csem).wait()
            pltpu.sync_copy(crow, acc_sh.at[cdi], add=True)

        def step(i, carry):
            @pl.when(i % 2 == 0)
            def _():
                halfstep(i, bufs[0], bufs[1])

            @pl.when(i % 2 == 1)
            def _():
                halfstep(i, bufs[1], bufs[0])

            return carry

        lax.fori_loop(0, NCH - 1, step, 0)
        last = bufs[(NCH - 1) % 2]
        pltpu.make_async_copy(x_hbm.at[last[0]], last[2], last[3]).wait()
        pltpu.sync_copy(last[2], acc_sh.at[last[1]], add=True)
        plsc.subcore_barrier()

        for k in range(RPT // CH):
            pltpu.sync_copy(acc_sh.at[pl.ds(s * RPT + k * CH, CH)], rows0)
            pltpu.sync_copy(rows0, acc_out.at[pl.ds(c * NPAD + s * RPT + k * CH, CH)])

    return pl.kernel(body, out_type=out_type, mesh=mesh,
                     scratch_types=scratch)


def _tc_layer1_body(acc0, acc1, cnt0, cnt1, x, wl, wr, b, out):
    cnt = cnt0[:, 0:1] + cnt1[:, 0:1]
    agg = (acc0[...] + acc1[...]) / jnp.maximum(cnt, 1.0)
    h = (jnp.dot(agg, wl[...], preferred_element_type=jnp.float32)
         + jnp.dot(x[...], wr[...], preferred_element_type=jnp.float32)
         + b[...])
    out[...] = jnp.maximum(h, 0.0)


def _tc_layer2_body(acc0, acc1, cnt0, cnt1, h1, batch, wl, wr, b,
                    wf1, bf1, wf2, bf2, out, pool_acc, n_acc):
    i = pl.program_id(0)

    @pl.when(i == 0)
    def _():
        pool_acc[...] = jnp.zeros_like(pool_acc)
        n_acc[...] = jnp.zeros_like(n_acc)

    cnt = cnt0[:, 0:1] + cnt1[:, 0:1]
    agg = (acc0[...] + acc1[...]) / jnp.maximum(cnt, 1.0)
    h = (jnp.dot(agg, wl[...], preferred_element_type=jnp.float32)
         + jnp.dot(h1[...], wr[...], preferred_element_type=jnp.float32)
         + b[...])
    h = jnp.maximum(h, 0.0)

    bt = batch[0, 0, :]
    gid = lax.broadcasted_iota(jnp.int32, (G, BLK), 0)
    onehot = (gid == bt[None, :]).astype(jnp.float32)
    pool_acc[...] += jnp.dot(onehot, h, preferred_element_type=jnp.float32)
    n_acc[...] += jnp.sum(onehot, axis=1, keepdims=True)

    @pl.when(i == NB - 1)
    def _():
        pooled = pool_acc[...] / jnp.maximum(n_acc[...], 1.0)
        hid = jnp.maximum(
            jnp.dot(pooled, wf1[...], preferred_element_type=jnp.float32)
            + bf1[...], 0.0)
        out[...] = (jnp.dot(hid, wf2[...], preferred_element_type=jnp.float32)
                    + bf2[...])


_row_spec = pl.BlockSpec((BLK, D), lambda i: (i, 0))
_cnt_spec = pl.BlockSpec((BLK, 16), lambda i: (i, 0))
_full = lambda shape: pl.BlockSpec(shape, lambda i: tuple(0 for _ in shape))

_tc_layer1 = pl.pallas_call(
    _tc_layer1_body,
    grid=(NB,),
    in_specs=[_row_spec, _row_spec, _cnt_spec, _cnt_spec, _row_spec,
              _full((D, H)), _full((D, H)), _full((1, H))],
    out_specs=_row_spec,
    out_shape=jax.ShapeDtypeStruct((NPAD, H), jnp.float32),
    compiler_params=pltpu.CompilerParams(
        dimension_semantics=("arbitrary",)),
)

_tc_layer2 = pl.pallas_call(
    _tc_layer2_body,
    grid=(NB,),
    in_specs=[_row_spec, _row_spec, _cnt_spec, _cnt_spec, _row_spec,
              pl.BlockSpec((1, 1, BLK), lambda i: (i, 0, 0)),
              _full((H, H)), _full((H, H)), _full((1, H)),
              _full((H, H // 2)), _full((1, H // 2)),
              _full((H // 2, 128)), _full((1, 128))],
    out_specs=_full((G, 128)),
    out_shape=jax.ShapeDtypeStruct((G, 128), jnp.float32),
    scratch_shapes=[pltpu.VMEM((G, H), jnp.float32),
                    pltpu.VMEM((G, H), jnp.float32)],
    compiler_params=pltpu.CompilerParams(
        dimension_semantics=("arbitrary",)),
)

_sc_agg = _make_sc_agg(D)


def _sc_cnt_body(dst_hbm, z_hbm, ones_hbm, cnt_out, didx, ones_v, acc_sh):
    c = lax.axis_index("c")
    s = lax.axis_index("s")
    wid = s * 2 + c

    pltpu.sync_copy(z_hbm.at[pl.ds(0, CH)], ones_v)
    for k in range(RPT // CH):
        pltpu.sync_copy(ones_v, acc_sh.at[pl.ds(s * RPT + k * CH, CH)])
    pltpu.sync_copy(ones_hbm, ones_v)
    plsc.subcore_barrier()

    base0 = wid * EPW

    def step(i, carry):
        pltpu.sync_copy(dst_hbm.at[pl.ds(base0 + i * CH, CH)], didx)
        pltpu.sync_copy(ones_v, acc_sh.at[didx], add=True)
        return carry

    lax.fori_loop(0, NCH, step, 0)
    plsc.subcore_barrier()

    for k in range(RPT // CH):
        pltpu.sync_copy(acc_sh.at[pl.ds(s * RPT + k * CH, CH)], ones_v)
        pltpu.sync_copy(ones_v, cnt_out.at[pl.ds(c * NPAD + s * RPT + k * CH, CH)])


_sc_cnt = pl.kernel(
    _sc_cnt_body,
    out_type=jax.ShapeDtypeStruct((2 * NPAD, D), jnp.float32),
    mesh=plsc.VectorSubcoreMesh(core_axis_name="c", subcore_axis_name="s"),
    scratch_types=[
        pltpu.VMEM((CH,), jnp.int32),
        pltpu.VMEM((CH, D), jnp.float32),
        pltpu.VMEM_SHARED((NPAD, D), jnp.float32),
    ],
)


def kernel(x, edge_index, batch, W1l, b1l, W1r, W2l, b2l, W2r,
           Wf1, bf1, Wf2, bf2):
    f32 = jnp.float32
    src_p = edge_index[0]
    dst_p = edge_index[1]
    x_p = jnp.pad(x, ((0, NPAD - N), (0, 0)))
    batch_p = jnp.pad(batch, (0, NPAD - N), constant_values=G)
    batch3 = batch_p.reshape(NB, 1, BLK)

    z2 = jnp.zeros((NPAD, D), f32)
    ones_rows = jnp.ones((CH, D), f32)

    acc = _sc_agg(x_p, src_p, dst_p, z2)
    cntw = _sc_cnt(dst_p, z2, ones_rows)
    cnt0, cnt1 = cntw[:NPAD, :16], cntw[NPAD:, :16]
    h1 = _tc_layer1(acc[:NPAD], acc[NPAD:], cnt0, cnt1, x_p,
                    W1l.T, W1r.T, b1l.reshape(1, H))

    acc2 = _sc_agg(h1, src_p, dst_p, z2)
    wf2_pad = jnp.zeros((H // 2, 128), f32).at[:, :C].set(Wf2.T)
    bf2_pad = jnp.zeros((1, 128), f32).at[0, :C].set(bf2)
    out_pad = _tc_layer2(acc2[:NPAD], acc2[NPAD:], cnt0, cnt1, h1, batch3,
                         W2l.T, W2r.T, b2l.reshape(1, H),
                         Wf1.T, bf1.reshape(1, H // 2), wf2_pad, bf2_pad)
    return out_pad[:, :C]


# speedup vs baseline: 8.3979x; 1.2619x over previous
import functools

import jax
import jax.numpy as jnp
from jax import lax
from jax.experimental import pallas as pl
from jax.experimental.pallas import tpu as pltpu
from jax.experimental.pallas import tpu_sc as plsc

N = 10000
E = 320000
D = 128
H = 128
G = 16
C = 2

NPAD = 10240
NW = 32
CH = 80
NCH = 125
EPW = CH * NCH
RPT = NPAD // 16
BLK = 512
NB = NPAD // BLK


def _make_sc_agg(W):
    out_type = jax.ShapeDtypeStruct((2 * NPAD, W), jnp.float32)
    scratch = [
        pltpu.VMEM((CH,), jnp.int32),
        pltpu.VMEM((CH,), jnp.int32),
        pltpu.VMEM((CH,), jnp.int32),
        pltpu.VMEM((CH,), jnp.int32),
        pltpu.VMEM((CH, W), jnp.float32),
        pltpu.VMEM((CH, W), jnp.float32),
        pltpu.VMEM_SHARED((NPAD, W), jnp.float32),
        pltpu.SemaphoreType.DMA,
        pltpu.SemaphoreType.DMA,
        pltpu.SemaphoreType.DMA,
        pltpu.SemaphoreType.DMA,
        pltpu.SemaphoreType.DMA,
    ]

    mesh = plsc.VectorSubcoreMesh(core_axis_name="c", subcore_axis_name="s")

    def body(x_hbm, src_hbm, dst_hbm, z2_hbm, acc_out, sidx0, sidx1,
             didx0, didx1, rows0, rows1, acc_sh, gsem0, gsem1,
             ssem0, ssem1, isem):
        c = lax.axis_index("c")
        s = lax.axis_index("s")
        wid = s * 2 + c

        pltpu.sync_copy(z2_hbm.at[pl.ds(0, CH)], rows0)
        for k in range(RPT // CH):
            pltpu.sync_copy(rows0, acc_sh.at[pl.ds(s * RPT + k * CH, CH)])
        plsc.subcore_barrier()

        base0 = wid * EPW
        bufs = ((sidx0, didx0, rows0, gsem0, ssem0),
                (sidx1, didx1, rows1, gsem1, ssem1))

        pltpu.sync_copy(src_hbm.at[pl.ds(base0, CH)], sidx0)
        pltpu.sync_copy(dst_hbm.at[pl.ds(base0, CH)], didx0)
        pltpu.async_copy(x_hbm.at[sidx0], rows0, gsem0)

        def halfstep(i, cur, nxt):
            csi, cdi, crow, cg, cs = cur
            nsi, ndi, nrow, ng, ns = nxt

            @pl.when(i > 0)
            def _():
                pltpu.make_async_copy(nrow, acc_sh.at[ndi], ns).wait()

            b = base0 + (i + 1) * CH
            pltpu.async_copy(src_hbm.at[pl.ds(b, CH)], nsi, isem)
            pltpu.async_copy(dst_hbm.at[pl.ds(b, CH)], ndi, isem)
            pltpu.make_async_copy(x_hbm.at[csi], crow, cg).wait()
            pltpu.make_async_copy(src_hbm.at[pl.ds(b, CH)], nsi, isem).wait()
            pltpu.make_async_copy(dst_hbm.at[pl.ds(b, CH)], ndi, isem).wait()
            pltpu.async_copy(x_hbm.at[nsi], nrow, ng)
            pltpu.async_copy(crow, acc_sh.at[cdi], cs, add=True)

        def step(i, carry):
            @pl.when(i % 2 == 0)
            def _():
                halfstep(i, bufs[0], bufs[1])

            @pl.when(i % 2 == 1)
            def _():
                halfstep(i, bufs[1], bufs[0])

            return carry

        lax.fori_loop(0, NCH - 1, step, 0)
        sl, ll = bufs[(NCH - 2) % 2], bufs[(NCH - 1) % 2]
        pltpu.make_async_copy(sl[2], acc_sh.at[sl[1]], sl[4]).wait()
        pltpu.make_async_copy(x_hbm.at[ll[0]], ll[2], ll[3]).wait()
        pltpu.sync_copy(ll[2], acc_sh.at[ll[1]], add=True)
        plsc.subcore_barrier()

        for k in range(RPT // CH):
            pltpu.sync_copy(acc_sh.at[pl.ds(s * RPT + k * CH, CH)], rows0)
            pltpu.sync_copy(rows0, acc_out.at[pl.ds(c * NPAD + s * RPT + k * CH, CH)])

    return pl.kernel(body, out_type=out_type, mesh=mesh,
                     scratch_types=scratch)


def _tc_layer1_body(acc0, acc1, cnt0, cnt1, x, wl, wr, b, out):
    cnt = cnt0[:, 0:1] + cnt1[:, 0:1]
    agg = (acc0[...] + acc1[...]) / jnp.maximum(cnt, 1.0)
    h = (jnp.dot(agg, wl[...], preferred_element_type=jnp.float32)
         + jnp.dot(x[...], wr[...], preferred_element_type=jnp.float32)
         + b[...])
    out[...] = jnp.maximum(h, 0.0)


def _tc_layer2_body(acc0, acc1, cnt0, cnt1, h1, batch, wl, wr, b,
                    wf1, bf1, wf2, bf2, out, pool_acc, n_acc):
    i = pl.program_id(0)

    @pl.when(i == 0)
    def _():
        pool_acc[...] = jnp.zeros_like(pool_acc)
        n_acc[...] = jnp.zeros_like(n_acc)

    cnt = cnt0[:, 0:1] + cnt1[:, 0:1]
    agg = (acc0[...] + acc1[...]) / jnp.maximum(cnt, 1.0)
    h = (jnp.dot(agg, wl[...], preferred_element_type=jnp.float32)
         + jnp.dot(h1[...], wr[...], preferred_element_type=jnp.float32)
         + b[...])
    h = jnp.maximum(h, 0.0)

    bt = batch[0, 0, :]
    gid = lax.broadcasted_iota(jnp.int32, (G, BLK), 0)
    onehot = (gid == bt[None, :]).astype(jnp.float32)
    pool_acc[...] += jnp.dot(onehot, h, preferred_element_type=jnp.float32)
    n_acc[...] += jnp.sum(onehot, axis=1, keepdims=True)

    @pl.when(i == NB - 1)
    def _():
        pooled = pool_acc[...] / jnp.maximum(n_acc[...], 1.0)
        hid = jnp.maximum(
            jnp.dot(pooled, wf1[...], preferred_element_type=jnp.float32)
            + bf1[...], 0.0)
        out[...] = (jnp.dot(hid, wf2[...], preferred_element_type=jnp.float32)
                    + bf2[...])


_row_spec = pl.BlockSpec((BLK, D), lambda i: (i, 0))
_row_spec_hi = pl.BlockSpec((BLK, D), lambda i: (i + NB, 0))
_full = lambda shape: pl.BlockSpec(shape, lambda i: tuple(0 for _ in shape))

_tc_layer1 = pl.pallas_call(
    _tc_layer1_body,
    grid=(NB,),
    in_specs=[_row_spec, _row_spec_hi, _row_spec, _row_spec_hi, _row_spec,
              _full((D, H)), _full((D, H)), _full((1, H))],
    out_specs=_row_spec,
    out_shape=jax.ShapeDtypeStruct((NPAD, H), jnp.float32),
    compiler_params=pltpu.CompilerParams(
        dimension_semantics=("arbitrary",)),
)

_tc_layer2 = pl.pallas_call(
    _tc_layer2_body,
    grid=(NB,),
    in_specs=[_row_spec, _row_spec_hi, _row_spec, _row_spec_hi, _row_spec,
              pl.BlockSpec((1, 1, BLK), lambda i: (i, 0, 0)),
              _full((H, H)), _full((H, H)), _full((1, H)),
              _full((H, H // 2)), _full((1, H // 2)),
              _full((H // 2, 128)), _full((1, 128))],
    out_specs=_full((G, 128)),
    out_shape=jax.ShapeDtypeStruct((G, 128), jnp.float32),
    scratch_shapes=[pltpu.VMEM((G, H), jnp.float32),
                    pltpu.VMEM((G, H), jnp.float32)],
    compiler_params=pltpu.CompilerParams(
        dimension_semantics=("arbitrary",)),
)

_sc_agg = _make_sc_agg(D)


def _sc_cnt_body(dst_hbm, z_hbm, ones_hbm, cnt_out, didx0, didx1, ones_v,
                 acc_sh, ssem0, ssem1, isem):
    c = lax.axis_index("c")
    s = lax.axis_index("s")
    wid = s * 2 + c

    pltpu.sync_copy(z_hbm.at[pl.ds(0, CH)], ones_v)
    for k in range(RPT // CH):
        pltpu.sync_copy(ones_v, acc_sh.at[pl.ds(s * RPT + k * CH, CH)])
    pltpu.sync_copy(ones_hbm, ones_v)
    plsc.subcore_barrier()

    base0 = wid * EPW
    bufs = ((didx0, ssem0), (didx1, ssem1))
    pltpu.sync_copy(dst_hbm.at[pl.ds(base0, CH)], didx0)

    def halfstep(i, cur, nxt):
        cdi, cs = cur
        ndi, ns = nxt

        @pl.when(i > 0)
        def _():
            pltpu.make_async_copy(ones_v, acc_sh.at[ndi], ns).wait()

        b = base0 + (i + 1) * CH
        pltpu.async_copy(dst_hbm.at[pl.ds(b, CH)], ndi, isem)
        pltpu.async_copy(ones_v, acc_sh.at[cdi], cs, add=True)
        pltpu.make_async_copy(dst_hbm.at[pl.ds(b, CH)], ndi, isem).wait()

    def step(i, carry):
        @pl.when(i % 2 == 0)
        def _():
            halfstep(i, bufs[0], bufs[1])

        @pl.when(i % 2 == 1)
        def _():
            halfstep(i, bufs[1], bufs[0])

        return carry

    lax.fori_loop(0, NCH - 1, step, 0)
    sl, ll = bufs[(NCH - 2) % 2], bufs[(NCH - 1) % 2]
    pltpu.make_async_copy(ones_v, acc_sh.at[sl[0]], sl[1]).wait()
    pltpu.sync_copy(ones_v, acc_sh.at[ll[0]], add=True)
    plsc.subcore_barrier()

    for k in range(RPT // CH):
        pltpu.sync_copy(acc_sh.at[pl.ds(s * RPT + k * CH, CH)], ones_v)
        pltpu.sync_copy(ones_v, cnt_out.at[pl.ds(c * NPAD + s * RPT + k * CH, CH)])


_sc_cnt = pl.kernel(
    _sc_cnt_body,
    out_type=jax.ShapeDtypeStruct((2 * NPAD, D), jnp.float32),
    mesh=plsc.VectorSubcoreMesh(core_axis_name="c", subcore_axis_name="s"),
    scratch_types=[
        pltpu.VMEM((CH,), jnp.int32),
        pltpu.VMEM((CH,), jnp.int32),
        pltpu.VMEM((CH, D), jnp.float32),
        pltpu.VMEM_SHARED((NPAD, D), jnp.float32),
        pltpu.SemaphoreType.DMA,
        pltpu.SemaphoreType.DMA,
        pltpu.SemaphoreType.DMA,
    ],
)


def kernel(x, edge_index, batch, W1l, b1l, W1r, W2l, b2l, W2r,
           Wf1, bf1, Wf2, bf2):
    f32 = jnp.float32
    src_p = edge_index[0]
    dst_p = edge_index[1]
    x_p = jnp.pad(x, ((0, NPAD - N), (0, 0)))
    batch_p = jnp.pad(batch, (0, NPAD - N), constant_values=G)
    batch3 = batch_p.reshape(NB, 1, BLK)

    z2 = jnp.zeros((NPAD, D), f32)
    ones_rows = jnp.ones((CH, D), f32)

    acc = _sc_agg(x_p, src_p, dst_p, z2)
    cntw = _sc_cnt(dst_p, z2, ones_rows)
    h1 = _tc_layer1(acc, acc, cntw, cntw, x_p,
                    W1l.T, W1r.T, b1l.reshape(1, H))

    acc2 = _sc_agg(h1, src_p, dst_p, z2)
    wf2_pad = jnp.zeros((H // 2, 128), f32).at[:, :C].set(Wf2.T)
    bf2_pad = jnp.zeros((1, 128), f32).at[0, :C].set(bf2)
    out_pad = _tc_layer2(acc2, acc2, cntw, cntw, h1, batch3,
                         W2l.T, W2r.T, b2l.reshape(1, H),
                         Wf1.T, bf1.reshape(1, H // 2), wf2_pad, bf2_pad)
    return out_pad[:, :C]


# speedup vs baseline: 10.0594x; 1.1978x over previous
import functools

import jax
import jax.numpy as jnp
from jax import lax
from jax.experimental import pallas as pl
from jax.experimental.pallas import tpu as pltpu
from jax.experimental.pallas import tpu_sc as plsc

N = 10000
E = 320000
D = 128
H = 128
G = 16
C = 2

NPAD = 10240
NW = 32
CH = 80
NCH = 125
EPW = CH * NCH
RPT = NPAD // 16
BLK = 512
NB = NPAD // BLK


def _make_sc_agg(W):
    out_type = jax.ShapeDtypeStruct((2 * NPAD, W), jnp.float32)
    scratch = [
        pltpu.VMEM((4, CH), jnp.int32),
        pltpu.VMEM((4, CH), jnp.int32),
        pltpu.VMEM((CH, W), jnp.float32),
        pltpu.VMEM((CH, W), jnp.float32),
        pltpu.VMEM_SHARED((NPAD, W), jnp.float32),
        pltpu.SemaphoreType.DMA,
        pltpu.SemaphoreType.DMA,
        pltpu.SemaphoreType.DMA,
        pltpu.SemaphoreType.DMA,
        pltpu.SemaphoreType.DMA,
    ]

    mesh = plsc.VectorSubcoreMesh(core_axis_name="c", subcore_axis_name="s")

    def body(x_hbm, src_hbm, dst_hbm, z2_hbm, acc_out, sidx, didx,
             rows0, rows1, acc_sh, gsem0, gsem1, ssem0, ssem1, isem):
        c = lax.axis_index("c")
        s = lax.axis_index("s")
        wid = s * 2 + c

        pltpu.sync_copy(z2_hbm.at[pl.ds(0, CH)], rows0)
        for k in range(RPT // CH):
            pltpu.sync_copy(rows0, acc_sh.at[pl.ds(s * RPT + k * CH, CH)])
        plsc.subcore_barrier()

        base0 = wid * EPW
        rws = (rows0, rows1)
        gss = (gsem0, gsem1)
        sss = (ssem0, ssem1)

        pltpu.sync_copy(src_hbm.at[pl.ds(base0, CH)], sidx.at[0])
        pltpu.sync_copy(dst_hbm.at[pl.ds(base0, CH)], didx.at[0])
        pltpu.sync_copy(src_hbm.at[pl.ds(base0 + CH, CH)], sidx.at[1])
        pltpu.sync_copy(dst_hbm.at[pl.ds(base0 + CH, CH)], didx.at[1])
        pltpu.async_copy(x_hbm.at[sidx.at[0]], rows0, gsem0)

        def substep(i, j):
            p = j % 2
            crow, nrow = rws[p], rws[1 - p]
            cg, ng = gss[p], gss[1 - p]
            cs, ns = sss[p], sss[1 - p]

            @pl.when(i > 0)
            def _():
                pltpu.make_async_copy(nrow, acc_sh.at[didx.at[(j + 3) % 4]],
                                      ns).wait()

            pltpu.async_copy(x_hbm.at[sidx.at[(j + 1) % 4]], nrow, ng)

            @pl.when(i + 2 < NCH)
            def _():
                b = base0 + (i + 2) * CH
                pltpu.async_copy(src_hbm.at[pl.ds(b, CH)],
                                 sidx.at[(j + 2) % 4], isem)
                pltpu.async_copy(dst_hbm.at[pl.ds(b, CH)],
                                 didx.at[(j + 2) % 4], isem)

            pltpu.make_async_copy(x_hbm.at[sidx.at[j]], crow, cg).wait()
            pltpu.async_copy(crow, acc_sh.at[didx.at[j]], cs, add=True)

            @pl.when(i + 2 < NCH)
            def _():
                b = base0 + (i + 2) * CH
                pltpu.make_async_copy(src_hbm.at[pl.ds(b, CH)],
                                      sidx.at[(j + 2) % 4], isem).wait()
                pltpu.make_async_copy(dst_hbm.at[pl.ds(b, CH)],
                                      didx.at[(j + 2) % 4], isem).wait()

        def step(i, carry):
            for j in range(4):
                @pl.when(i % 4 == j)
                def _(j=j):
                    substep(i, j)

            return carry

        lax.fori_loop(0, NCH - 1, step, 0)
        lp = (NCH - 1) % 2
        lj = (NCH - 1) % 4
        pltpu.make_async_copy(rws[1 - lp], acc_sh.at[didx.at[(lj + 3) % 4]],
                              sss[1 - lp]).wait()
        pltpu.make_async_copy(x_hbm.at[sidx.at[lj]], rws[lp], gss[lp]).wait()
        pltpu.sync_copy(rws[lp], acc_sh.at[didx.at[lj]], add=True)
        plsc.subcore_barrier()

        for k in range(RPT // CH):
            pltpu.sync_copy(acc_sh.at[pl.ds(s * RPT + k * CH, CH)], rows0)
            pltpu.sync_copy(rows0, acc_out.at[pl.ds(c * NPAD + s * RPT + k * CH, CH)])

    return pl.kernel(body, out_type=out_type, mesh=mesh,
                     scratch_types=scratch)


def _tc_layer1_body(acc0, acc1, cnt0, cnt1, x, wl, wr, b, out):
    cnt = cnt0[:, 0:1] + cnt1[:, 0:1]
    agg = (acc0[...] + acc1[...]) / jnp.maximum(cnt, 1.0)
    h = (jnp.dot(agg, wl[...], preferred_element_type=jnp.float32)
         + jnp.dot(x[...], wr[...], preferred_element_type=jnp.float32)
         + b[...])
    out[...] = jnp.maximum(h, 0.0)


def _tc_layer2_body(acc0, acc1, cnt0, cnt1, h1, batch, wl, wr, b,
                    wf1, bf1, wf2, bf2, out, pool_acc, n_acc):
    i = pl.program_id(0)

    @pl.when(i == 0)
    def _():
        pool_acc[...] = jnp.zeros_like(pool_acc)
        n_acc[...] = jnp.zeros_like(n_acc)

    cnt = cnt0[:, 0:1] + cnt1[:, 0:1]
    agg = (acc0[...] + acc1[...]) / jnp.maximum(cnt, 1.0)
    h = (jnp.dot(agg, wl[...], preferred_element_type=jnp.float32)
         + jnp.dot(h1[...], wr[...], preferred_element_type=jnp.float32)
         + b[...])
    h = jnp.maximum(h, 0.0)

    bt = batch[0, 0, :]
    gid = lax.broadcasted_iota(jnp.int32, (G, BLK), 0)
    onehot = (gid == bt[None, :]).astype(jnp.float32)
    pool_acc[...] += jnp.dot(onehot, h, preferred_element_type=jnp.float32)
    n_acc[...] += jnp.sum(onehot, axis=1, keepdims=True)

    @pl.when(i == NB - 1)
    def _():
        pooled = pool_acc[...] / jnp.maximum(n_acc[...], 1.0)
        hid = jnp.maximum(
            jnp.dot(pooled, wf1[...], preferred_element_type=jnp.float32)
            + bf1[...], 0.0)
        out[...] = (jnp.dot(hid, wf2[...], preferred_element_type=jnp.float32)
                    + bf2[...])


_row_spec = pl.BlockSpec((BLK, D), lambda i: (i, 0))
_row_spec_hi = pl.BlockSpec((BLK, D), lambda i: (i + NB, 0))
_full = lambda shape: pl.BlockSpec(shape, lambda i: tuple(0 for _ in shape))

_tc_layer1 = pl.pallas_call(
    _tc_layer1_body,
    grid=(NB,),
    in_specs=[_row_spec, _row_spec_hi, _row_spec, _row_spec_hi, _row_spec,
              _full((D, H)), _full((D, H)), _full((1, H))],
    out_specs=_row_spec,
    out_shape=jax.ShapeDtypeStruct((NPAD, H), jnp.float32),
    compiler_params=pltpu.CompilerParams(
        dimension_semantics=("arbitrary",)),
)

_tc_layer2 = pl.pallas_call(
    _tc_layer2_body,
    grid=(NB,),
    in_specs=[_row_spec, _row_spec_hi, _row_spec, _row_spec_hi, _row_spec,
              pl.BlockSpec((1, 1, BLK), lambda i: (i, 0, 0)),
              _full((H, H)), _full((H, H)), _full((1, H)),
              _full((H, H // 2)), _full((1, H // 2)),
              _full((H // 2, 128)), _full((1, 128))],
    out_specs=_full((G, 128)),
    out_shape=jax.ShapeDtypeStruct((G, 128), jnp.float32),
    scratch_shapes=[pltpu.VMEM((G, H), jnp.float32),
                    pltpu.VMEM((G, H), jnp.float32)],
    compiler_params=pltpu.CompilerParams(
        dimension_semantics=("arbitrary",)),
)

_sc_agg = _make_sc_agg(D)


def _sc_cnt_body(dst_hbm, z_hbm, ones_hbm, cnt_out, didx0, didx1, ones_v,
                 acc_sh, ssem0, ssem1, isem):
    c = lax.axis_index("c")
    s = lax.axis_index("s")
    wid = s * 2 + c

    pltpu.sync_copy(z_hbm.at[pl.ds(0, CH)], ones_v)
    for k in range(RPT // CH):
        pltpu.sync_copy(ones_v, acc_sh.at[pl.ds(s * RPT + k * CH, CH)])
    pltpu.sync_copy(ones_hbm, ones_v)
    plsc.subcore_barrier()

    base0 = wid * EPW
    bufs = ((didx0, ssem0), (didx1, ssem1))
    pltpu.sync_copy(dst_hbm.at[pl.ds(base0, CH)], didx0)

    def halfstep(i, cur, nxt):
        cdi, cs = cur
        ndi, ns = nxt

        @pl.when(i > 0)
        def _():
            pltpu.make_async_copy(ones_v, acc_sh.at[ndi], ns).wait()

        b = base0 + (i + 1) * CH
        pltpu.async_copy(dst_hbm.at[pl.ds(b, CH)], ndi, isem)
        pltpu.async_copy(ones_v, acc_sh.at[cdi], cs, add=True)
        pltpu.make_async_copy(dst_hbm.at[pl.ds(b, CH)], ndi, isem).wait()

    def step(i, carry):
        @pl.when(i % 2 == 0)
        def _():
            halfstep(i, bufs[0], bufs[1])

        @pl.when(i % 2 == 1)
        def _():
            halfstep(i, bufs[1], bufs[0])

        return carry

    lax.fori_loop(0, NCH - 1, step, 0)
    sl, ll = bufs[(NCH - 2) % 2], bufs[(NCH - 1) % 2]
    pltpu.make_async_copy(ones_v, acc_sh.at[sl[0]], sl[1]).wait()
    pltpu.sync_copy(ones_v, acc_sh.at[ll[0]], add=True)
    plsc.subcore_barrier()

    for k in range(RPT // CH):
        pltpu.sync_copy(acc_sh.at[pl.ds(s * RPT + k * CH, CH)], ones_v)
        pltpu.sync_copy(ones_v, cnt_out.at[pl.ds(c * NPAD + s * RPT + k * CH, CH)])


_sc_cnt = pl.kernel(
    _sc_cnt_body,
    out_type=jax.ShapeDtypeStruct((2 * NPAD, D), jnp.float32),
    mesh=plsc.VectorSubcoreMesh(core_axis_name="c", subcore_axis_name="s"),
    scratch_types=[
        pltpu.VMEM((CH,), jnp.int32),
        pltpu.VMEM((CH,), jnp.int32),
        pltpu.VMEM((CH, D), jnp.float32),
        pltpu.VMEM_SHARED((NPAD, D), jnp.float32),
        pltpu.SemaphoreType.DMA,
        pltpu.SemaphoreType.DMA,
        pltpu.SemaphoreType.DMA,
    ],
)


def kernel(x, edge_index, batch, W1l, b1l, W1r, W2l, b2l, W2r,
           Wf1, bf1, Wf2, bf2):
    f32 = jnp.float32
    src_p = edge_index[0]
    dst_p = edge_index[1]
    x_p = jnp.pad(x, ((0, NPAD - N), (0, 0)))
    batch_p = jnp.pad(batch, (0, NPAD - N), constant_values=G)
    batch3 = batch_p.reshape(NB, 1, BLK)

    z2 = jnp.zeros((NPAD, D), f32)
    ones_rows = jnp.ones((CH, D), f32)

    acc = _sc_agg(x_p, src_p, dst_p, z2)
    cntw = _sc_cnt(dst_p, z2, ones_rows)
    h1 = _tc_layer1(acc, acc, cntw, cntw, x_p,
                    W1l.T, W1r.T, b1l.reshape(1, H))

    acc2 = _sc_agg(h1, src_p, dst_p, z2)
    wf2_pad = jnp.zeros((H // 2, 128), f32).at[:, :C].set(Wf2.T)
    bf2_pad = jnp.zeros((1, 128), f32).at[0, :C].set(bf2)
    out_pad = _tc_layer2(acc2, acc2, cntw, cntw, h1, batch3,
                         W2l.T, W2r.T, b2l.reshape(1, H),
                         Wf1.T, bf1.reshape(1, H // 2), wf2_pad, bf2_pad)
    return out_pad[:, :C]
